# baseline (speedup 1.0000x reference)
import functools

import jax
import jax.numpy as jnp
from jax import lax
from jax.experimental import pallas as pl
from jax.experimental.pallas import tpu as pltpu
from jax.experimental.pallas import tpu_sc as plsc

B = 16384
D = 32
WD = 1e-4

_NC = 2
_NS = 16
_NW = _NC * _NS
_L = 16

_VH = 1000000
_VW = 100000
_HFULL = (_VH // 128) * 128
_WFULL = (_VW // 128) * 128
_CWH = 245 * 128
_CWW = 25 * 128
_TCH = 8
_SLAB = _TCH * 128
_NCH_H = 31
_NCH_W = 4
_WLCAP = 1056
_WLBUF = _WLCAP + _L
_OUTR = B + _NW


def _sc_gather(u, i, j, Wmlp3, Hmlp3, Wmf3, Hmf3,
               Wmlp_t, Hmlp_t, Wmf_t, Hmf_t):
    mesh = plsc.VectorSubcoreMesh(core_axis_name="c", subcore_axis_name="s")
    out_t = tuple(
        jax.ShapeDtypeStruct((_OUTR, 128), jnp.float32) for _ in range(6))

    @functools.partial(
        pl.kernel,
        mesh=mesh,
        compiler_params=pltpu.CompilerParams(needs_layout_passes=False),
        out_type=out_t,
        scratch_types=[
            pltpu.VMEM((B,), jnp.int32),
            pltpu.VMEM((_WLBUF,), jnp.int32),
            pltpu.VMEM((_WLBUF,), jnp.int32),
            pltpu.VMEM((_WLBUF,), jnp.int32),
            pltpu.VMEM((_WLBUF,), jnp.int32),
            pltpu.VMEM((_WLBUF,), jnp.int32),
            pltpu.VMEM((_WLBUF,), jnp.int32),
            pltpu.VMEM((_WLBUF,), jnp.int32),
            pltpu.VMEM((_WLBUF,), jnp.int32),
            pltpu.VMEM((32, _SLAB), jnp.float32),
            pltpu.VMEM((32, _SLAB), jnp.float32),
            pltpu.VMEM((32, 128), jnp.float32),
            pltpu.VMEM((1, 32), jnp.int32),
            pltpu.SemaphoreType.DMA,
            pltpu.SemaphoreType.DMA,
            pltpu.SemaphoreType.DMA,
        ],
    )
    def gather_kernel(u_hbm, i_hbm, j_hbm, wmlp, hmlp, wmf, hmf,
                      wmlp_t, hmlp_t, wmf_t, hmf_t,
                      o_umlp, o_imlp, o_jmlp, o_umf, o_imf, o_jmf,
                      idxbuf, wur, wup, wir, wip, wjr, wjp, clr, clp,
                      slab_a, slab_b, staging, sidx, sem_a, sem_b, sem_s):
        wid = lax.axis_index("s") * _NC + lax.axis_index("c")
        dump = B + wid
        iota = lax.iota(jnp.int32, _L)

        def _sp(x):
            return jnp.full((_L,), x, jnp.int32)

        iota_lo = iota
        iota_hi = iota + _sp(_L)

        def compact_append(wr, wp, off, v, pos, m):
            pref = plsc.cumsum(m.astype(jnp.int32))
            tgt = jnp.where(m, _sp(off - 1) + pref, _sp(_WLCAP) + iota)
            plsc.store_scatter(wr, [tgt], v)
            plsc.store_scatter(wp, [tgt], pos)
            return off + pref[_L - 1]

        def scan(idx_hbm, lo, hi, wr, wp):
            pltpu.sync_copy(idx_hbm, idxbuf)
            lov = _sp(lo)
            hiv = _sp(hi)

            def body(k, off):
                for sub in range(4):
                    base = k * 64 + sub * _L
                    v = idxbuf[pl.ds(base, _L)]
                    pos = _sp(base) + iota
                    m = (v >= lov) & (v < hiv)
                    off = compact_append(wr, wp, off, v, pos, m)
                return off

            return lax.fori_loop(0, B // 64, body, jnp.int32(0))

        n_u = scan(u_hbm, wid * _CWW, (wid + 1) * _CWW, wur, wup)
        n_i = scan(i_hbm, wid * _CWH, (wid + 1) * _CWH, wir, wip)
        n_j = scan(j_hbm, wid * _CWH, (wid + 1) * _CWH, wjr, wjp)

        def extract(src, lo, width, wr, wp, n, out_hbm):
            lov = _sp(lo)
            nvv = _sp(n)
            wiv = _sp(width)
            zv = _sp(0)

            def cbody(k, coff):
                for sub in range(2):
                    kk = k * 2 + sub
                    r = wr[pl.ds(kk * _L, _L)]
                    p = wp[pl.ds(kk * _L, _L)]
                    valid = (_sp(kk * _L) + iota) < nvv
                    cc = r - lov
                    m = valid & (cc >= zv) & (cc < wiv)
                    coff = compact_append(clr, clp, coff, r, p, m)
                return coff

            nv = (n + 31) // 32
            cnt = lax.fori_loop(0, nv, cbody, jnp.int32(0))

            cntv = _sp(cnt)
            dmpv = _sp(dump)
            wm1v = _sp(width - 1)

            def ebody(e, _):
                for sub in range(2):
                    ee = e * 2 + sub
                    r = clr[pl.ds(ee * _L, _L)]
                    p = clp[pl.ds(ee * _L, _L)]
                    valid = (_sp(ee * _L) + iota) < cntv
                    cc = jnp.minimum(jnp.maximum(r - lov, zv), wm1v)
                    oidx = jnp.where(valid, p, dmpv)
                    sidx[0, pl.ds(sub * _L, _L)] = oidx
                    for s in range(_L):
                        col = jnp.full((_L,), cc[s], jnp.int32)
                        row = sub * _L + s
                        staging[row, pl.ds(0, _L)] = plsc.load_gather(
                            src, [iota_lo, col])
                        staging[row, pl.ds(_L, _L)] = plsc.load_gather(
                            src, [iota_hi, col])
                pltpu.async_copy(staging, out_hbm.at[sidx.at[0]],
                                 sem_s).wait()
                return 0

            ne = (cnt + 31) // 32
            lax.fori_loop(0, ne, ebody, 0)

        def stream_table(v3, tail_hbm, jobs, cw_tiles, nchunks, full, vfull):
            ntf = full // 128

            def lo_of(c):
                return pl.multiple_of(
                    jnp.minimum(wid * cw_tiles + c * _TCH, ntf - _TCH) * 128,
                    128)

            def issue(c, buf, sem):
                lo = lo_of(c)
                for g in range(4):
                    pltpu.async_copy(v3.at[g, :, pl.ds(lo, _SLAB)],
                                     buf.at[pl.ds(g * 8, 8)], sem)

            def drain(buf, sem):
                for g in range(4):
                    pltpu.make_async_copy(
                        v3.at[g, :, pl.ds(0, _SLAB)],
                        buf.at[pl.ds(g * 8, 8)], sem).wait()

            issue(0, slab_a, sem_a)
            npairs = (nchunks + 1) // 2

            def body(k, _):
                c0 = 2 * k
                c1 = 2 * k + 1

                @pl.when(c1 < nchunks)
                def _():
                    issue(c1, slab_b, sem_b)

                drain(slab_a, sem_a)
                for (wr, wp, n, out_hbm) in jobs:
                    extract(slab_a, lo_of(c0), _SLAB, wr, wp, n, out_hbm)

                @pl.when(c1 + 1 < nchunks)
                def _():
                    issue(c1 + 1, slab_a, sem_a)

                @pl.when(c1 < nchunks)
                def _():
                    drain(slab_b, sem_b)
                    for (wr, wp, n, out_hbm) in jobs:
                        extract(slab_b, lo_of(c1), _SLAB, wr, wp, n, out_hbm)

                return 0

            lax.fori_loop(0, npairs, body, 0)

            pltpu.sync_copy(tail_hbm, slab_a.at[:, pl.ds(0, 128)])
            for (wr, wp, n, out_hbm) in jobs:
                extract(slab_a, vfull - 128, 128, wr, wp, n, out_hbm)

        stream_table(wmlp, wmlp_t, [(wur, wup, n_u, o_umlp)],
                     _CWW // 128, _NCH_W, _WFULL, _VW)
        stream_table(wmf, wmf_t, [(wur, wup, n_u, o_umf)],
                     _CWW // 128, _NCH_W, _WFULL, _VW)
        stream_table(hmlp, hmlp_t,
                     [(wir, wip, n_i, o_imlp), (wjr, wjp, n_j, o_jmlp)],
                     _CWH // 128, _NCH_H, _HFULL, _VH)
        stream_table(hmf, hmf_t,
                     [(wir, wip, n_i, o_imf), (wjr, wjp, n_j, o_jmf)],
                     _CWH // 128, _NCH_H, _HFULL, _VH)

    return gather_kernel(u, i, j, Wmlp3, Hmlp3, Wmf3, Hmf3,
                         Wmlp_t, Hmlp_t, Wmf_t, Hmf_t)


_BLK = 2048


def _tc_head_kernel(ue_ref, ie_ref, je_ref, uef_ref, ief_ref, jef_ref,
                    fc0t_ref, fc0b_ref, fc1t_ref, fc1b_ref,
                    afft_ref, affb_ref, out_ref):
    ue = ue_ref[:, :D]
    ie = ie_ref[:, :D]
    je = je_ref[:, :D]
    uef = uef_ref[:, :D]
    ief = ief_ref[:, :D]
    jef = jef_ref[:, :D]
    fc0t = fc0t_ref[...]
    a0u = fc0t[:D]
    a0i = fc0t[D:]
    fc1t = fc1t_ref[...]
    afft = afft_ref[...]
    b0 = fc0b_ref[...]
    b1 = fc1b_ref[...]

    def head(item_mlp, item_mf):
        h0 = jnp.maximum(
            jnp.dot(ue, a0u, preferred_element_type=jnp.float32)
            + jnp.dot(item_mlp, a0i, preferred_element_type=jnp.float32)
            + b0, 0.0)
        h1 = jnp.maximum(
            jnp.dot(h0, fc1t, preferred_element_type=jnp.float32) + b1, 0.0)
        mf = uef * item_mf
        logit = (jnp.dot(h1, afft[:16], preferred_element_type=jnp.float32)
                 + jnp.dot(mf, afft[16:], preferred_element_type=jnp.float32))
        return logit[:, 0] + affb_ref[0, 0]

    x = head(ie, ief) - head(je, jef)
    neg_log_prob = jnp.maximum(-x, 0.0) + jnp.log1p(jnp.exp(-jnp.abs(x)))

    def nrm(a):
        return jnp.sqrt(jnp.sum(a * a, axis=1))

    reg = WD * (nrm(ue) + nrm(uef) + nrm(ie) + nrm(ief) + nrm(je) + nrm(jef))
    out_ref[...] = neg_log_prob + reg


def _tc_head(gu_mlp, gi_mlp, gj_mlp, gu_mf, gi_mf, gj_mf,
             fc0t, fc0b, fc1t, fc1b, afft, affb):
    row_spec = pl.BlockSpec((_BLK, 128), lambda b: (b, 0))

    def full(shape):
        return pl.BlockSpec(shape, lambda b, _n=len(shape): (0,) * _n)

    return pl.pallas_call(
        _tc_head_kernel,
        grid=(B // _BLK,),
        in_specs=[row_spec] * 6 + [
            full((64, D)), full((1, D)), full((D, 16)), full((1, 16)),
            full((48, 1)), full((1, 1)),
        ],
        out_specs=pl.BlockSpec((_BLK,), lambda b: (b,)),
        out_shape=jax.ShapeDtypeStruct((B,), jnp.float32),
    )(gu_mlp, gi_mlp, gj_mlp, gu_mf, gi_mf, gj_mf,
      fc0t, fc0b, fc1t, fc1b, afft, affb)


def kernel(u, i, j, W_mlp, H_mlp, W_mf, H_mf,
           fc0_w, fc0_b, fc1_w, fc1_b, aff_w, aff_b):
    Wmlp3 = W_mlp.T.reshape(4, 8, _VW)
    Hmlp3 = H_mlp.T.reshape(4, 8, _VH)
    Wmf3 = W_mf.T.reshape(4, 8, _VW)
    Hmf3 = H_mf.T.reshape(4, 8, _VH)
    Wmlp_t = W_mlp.T[:, _VW - 128:]
    Hmlp_t = H_mlp.T[:, _VH - 128:]
    Wmf_t = W_mf.T[:, _VW - 128:]
    Hmf_t = H_mf.T[:, _VH - 128:]
    outs = _sc_gather(u, i, j, Wmlp3, Hmlp3, Wmf3, Hmf3,
                      Wmlp_t, Hmlp_t, Wmf_t, Hmf_t)
    return _tc_head(
        *outs,
        fc0_w.T, fc0_b.reshape(1, D),
        fc1_w.T, fc1_b.reshape(1, 16),
        aff_w.T, aff_b.reshape(1, 1))


# speedup vs baseline: 1.2110x; 1.2110x over previous
import functools

import jax
import jax.numpy as jnp
from jax import lax
from jax.experimental import pallas as pl
from jax.experimental.pallas import tpu as pltpu
from jax.experimental.pallas import tpu_sc as plsc

B = 16384
D = 32
WD = 1e-4

_NC = 2
_NS = 16
_NW = _NC * _NS
_L = 16

_VH = 1000000
_VW = 100000
_HFULL = (_VH // 128) * 128
_WFULL = (_VW // 128) * 128
_CWH = 245 * 128
_CWW = 25 * 128
_TCH = 8
_SLAB = _TCH * 128
_NCH_H = 31
_NCH_W = 4
_WLCAP = 1056
_WLBUF = _WLCAP + _L
_OUTR = B + _NW


def _sc_gather(u, i, j, Wmlp3, Hmlp3, Wmf3, Hmf3,
               Wmlp_t, Hmlp_t, Wmf_t, Hmf_t):
    mesh = plsc.VectorSubcoreMesh(core_axis_name="c", subcore_axis_name="s")
    out_t = tuple(
        jax.ShapeDtypeStruct((_OUTR, 128), jnp.float32) for _ in range(6))

    @functools.partial(
        pl.kernel,
        mesh=mesh,
        compiler_params=pltpu.CompilerParams(needs_layout_passes=False),
        out_type=out_t,
        scratch_types=[
            pltpu.VMEM((B,), jnp.int32),
            pltpu.VMEM((_WLBUF,), jnp.int32),
            pltpu.VMEM((_WLBUF,), jnp.int32),
            pltpu.VMEM((_WLBUF,), jnp.int32),
            pltpu.VMEM((_WLBUF,), jnp.int32),
            pltpu.VMEM((_WLBUF,), jnp.int32),
            pltpu.VMEM((_WLBUF,), jnp.int32),
            pltpu.VMEM((_WLBUF,), jnp.int32),
            pltpu.VMEM((_WLBUF,), jnp.int32),
            pltpu.VMEM((32, _SLAB), jnp.float32),
            pltpu.VMEM((32, _SLAB), jnp.float32),
            pltpu.VMEM((_L, 128), jnp.float32),
            pltpu.VMEM((1, _L), jnp.int32),
            pltpu.SemaphoreType.DMA,
            pltpu.SemaphoreType.DMA,
            pltpu.SemaphoreType.DMA,
        ],
    )
    def gather_kernel(u_hbm, i_hbm, j_hbm, wmlp, hmlp, wmf, hmf,
                      wmlp_t, hmlp_t, wmf_t, hmf_t,
                      o_umlp, o_imlp, o_jmlp, o_umf, o_imf, o_jmf,
                      idxbuf, wur, wup, wir, wip, wjr, wjp, clr, clp,
                      slab_a, slab_b, staging, sidx, sem_a, sem_b, sem_s):
        wid = lax.axis_index("s") * _NC + lax.axis_index("c")
        dump = B + wid
        iota = lax.iota(jnp.int32, _L)

        def _sp(x):
            return jnp.full((_L,), x, jnp.int32)

        iota_lo = iota
        iota_hi = iota + _sp(_L)

        def compact_append(wr, wp, off, v, pos, m):
            pref = plsc.cumsum(m.astype(jnp.int32))
            tgt = jnp.where(m, _sp(off - 1) + pref, _sp(_WLCAP) + iota)
            plsc.store_scatter(wr, [tgt], v)
            plsc.store_scatter(wp, [tgt], pos)
            return off + pref[_L - 1]

        def scan(idx_hbm, lo, hi, wr, wp):
            pltpu.sync_copy(idx_hbm, idxbuf)

            def body(k, off):
                v = idxbuf[pl.ds(k * _L, _L)]
                pos = _sp(k * _L) + iota
                m = (v >= _sp(lo)) & (v < _sp(hi))
                return compact_append(wr, wp, off, v, pos, m)

            return lax.fori_loop(0, B // _L, body, jnp.int32(0))

        n_u = scan(u_hbm, wid * _CWW, (wid + 1) * _CWW, wur, wup)
        n_i = scan(i_hbm, wid * _CWH, (wid + 1) * _CWH, wir, wip)
        n_j = scan(j_hbm, wid * _CWH, (wid + 1) * _CWH, wjr, wjp)

        def extract(src, lo, width, wr, wp, n, out_hbm):
            def cbody(k, coff):
                r = wr[pl.ds(k * _L, _L)]
                p = wp[pl.ds(k * _L, _L)]
                valid = (_sp(k * _L) + iota) < _sp(n)
                cc = r - _sp(lo)
                m = valid & (cc >= _sp(0)) & (cc < _sp(width))
                return compact_append(clr, clp, coff, r, p, m)

            nv = (n + _L - 1) // _L
            cnt = lax.fori_loop(0, nv, cbody, jnp.int32(0))

            def ebody(e, _):
                r = clr[pl.ds(e * _L, _L)]
                p = clp[pl.ds(e * _L, _L)]
                valid = (_sp(e * _L) + iota) < _sp(cnt)
                cc = jnp.minimum(
                    jnp.maximum(r - _sp(lo), _sp(0)), _sp(width - 1))
                oidx = jnp.where(valid, p, _sp(dump))
                for s in range(_L):
                    col = jnp.full((_L,), cc[s], jnp.int32)
                    staging[s, pl.ds(0, _L)] = plsc.load_gather(
                        src, [iota_lo, col])
                    staging[s, pl.ds(_L, _L)] = plsc.load_gather(
                        src, [iota_hi, col])
                sidx[0, :] = oidx
                pltpu.async_copy(staging, out_hbm.at[sidx.at[0]],
                                 sem_s).wait()
                return 0

            ne = (cnt + _L - 1) // _L
            lax.fori_loop(0, ne, ebody, 0)

        def stream_table(v3, tail_hbm, jobs, cw_tiles, nchunks, full, vfull):
            ntf = full // 128

            def lo_of(c):
                return pl.multiple_of(
                    jnp.minimum(wid * cw_tiles + c * _TCH, ntf - _TCH) * 128,
                    128)

            def issue(c, buf, sem):
                lo = lo_of(c)
                for g in range(4):
                    pltpu.async_copy(v3.at[g, :, pl.ds(lo, _SLAB)],
                                     buf.at[pl.ds(g * 8, 8)], sem)

            def drain(buf, sem):
                for g in range(4):
                    pltpu.make_async_copy(
                        v3.at[g, :, pl.ds(0, _SLAB)],
                        buf.at[pl.ds(g * 8, 8)], sem).wait()

            issue(0, slab_a, sem_a)
            npairs = (nchunks + 1) // 2

            def body(k, _):
                c0 = 2 * k
                c1 = 2 * k + 1

                @pl.when(c1 < nchunks)
                def _():
                    issue(c1, slab_b, sem_b)

                drain(slab_a, sem_a)
                for (wr, wp, n, out_hbm) in jobs:
                    extract(slab_a, lo_of(c0), _SLAB, wr, wp, n, out_hbm)

                @pl.when(c1 + 1 < nchunks)
                def _():
                    issue(c1 + 1, slab_a, sem_a)

                @pl.when(c1 < nchunks)
                def _():
                    drain(slab_b, sem_b)
                    for (wr, wp, n, out_hbm) in jobs:
                        extract(slab_b, lo_of(c1), _SLAB, wr, wp, n, out_hbm)

                return 0

            lax.fori_loop(0, npairs, body, 0)

            pltpu.sync_copy(tail_hbm, slab_a.at[:, pl.ds(0, 128)])
            for (wr, wp, n, out_hbm) in jobs:
                extract(slab_a, vfull - 128, 128, wr, wp, n, out_hbm)

        stream_table(wmlp, wmlp_t, [(wur, wup, n_u, o_umlp)],
                     _CWW // 128, _NCH_W, _WFULL, _VW)
        stream_table(wmf, wmf_t, [(wur, wup, n_u, o_umf)],
                     _CWW // 128, _NCH_W, _WFULL, _VW)
        stream_table(hmlp, hmlp_t,
                     [(wir, wip, n_i, o_imlp), (wjr, wjp, n_j, o_jmlp)],
                     _CWH // 128, _NCH_H, _HFULL, _VH)
        stream_table(hmf, hmf_t,
                     [(wir, wip, n_i, o_imf), (wjr, wjp, n_j, o_jmf)],
                     _CWH // 128, _NCH_H, _HFULL, _VH)

    return gather_kernel(u, i, j, Wmlp3, Hmlp3, Wmf3, Hmf3,
                         Wmlp_t, Hmlp_t, Wmf_t, Hmf_t)


_BLK = 2048


def _tc_head_kernel(ue_ref, ie_ref, je_ref, uef_ref, ief_ref, jef_ref,
                    fc0t_ref, fc0b_ref, fc1t_ref, fc1b_ref,
                    afft_ref, affb_ref, out_ref):
    ue = ue_ref[:, :D]
    ie = ie_ref[:, :D]
    je = je_ref[:, :D]
    uef = uef_ref[:, :D]
    ief = ief_ref[:, :D]
    jef = jef_ref[:, :D]
    fc0t = fc0t_ref[...]
    a0u = fc0t[:D]
    a0i = fc0t[D:]
    fc1t = fc1t_ref[...]
    afft = afft_ref[...]
    b0 = fc0b_ref[...]
    b1 = fc1b_ref[...]

    def head(item_mlp, item_mf):
        h0 = jnp.maximum(
            jnp.dot(ue, a0u, preferred_element_type=jnp.float32)
            + jnp.dot(item_mlp, a0i, preferred_element_type=jnp.float32)
            + b0, 0.0)
        h1 = jnp.maximum(
            jnp.dot(h0, fc1t, preferred_element_type=jnp.float32) + b1, 0.0)
        mf = uef * item_mf
        logit = (jnp.dot(h1, afft[:16], preferred_element_type=jnp.float32)
                 + jnp.dot(mf, afft[16:], preferred_element_type=jnp.float32))
        return logit[:, 0] + affb_ref[0, 0]

    x = head(ie, ief) - head(je, jef)
    neg_log_prob = jnp.maximum(-x, 0.0) + jnp.log1p(jnp.exp(-jnp.abs(x)))

    def nrm(a):
        return jnp.sqrt(jnp.sum(a * a, axis=1))

    reg = WD * (nrm(ue) + nrm(uef) + nrm(ie) + nrm(ief) + nrm(je) + nrm(jef))
    out_ref[...] = neg_log_prob + reg


def _tc_head(gu_mlp, gi_mlp, gj_mlp, gu_mf, gi_mf, gj_mf,
             fc0t, fc0b, fc1t, fc1b, afft, affb):
    row_spec = pl.BlockSpec((_BLK, 128), lambda b: (b, 0))

    def full(shape):
        return pl.BlockSpec(shape, lambda b, _n=len(shape): (0,) * _n)

    return pl.pallas_call(
        _tc_head_kernel,
        grid=(B // _BLK,),
        in_specs=[row_spec] * 6 + [
            full((64, D)), full((1, D)), full((D, 16)), full((1, 16)),
            full((48, 1)), full((1, 1)),
        ],
        out_specs=pl.BlockSpec((_BLK,), lambda b: (b,)),
        out_shape=jax.ShapeDtypeStruct((B,), jnp.float32),
    )(gu_mlp, gi_mlp, gj_mlp, gu_mf, gi_mf, gj_mf,
      fc0t, fc0b, fc1t, fc1b, afft, affb)


def kernel(u, i, j, W_mlp, H_mlp, W_mf, H_mf,
           fc0_w, fc0_b, fc1_w, fc1_b, aff_w, aff_b):
    Wmlp3 = W_mlp.T.reshape(4, 8, _VW)
    Hmlp3 = H_mlp.T.reshape(4, 8, _VH)
    Wmf3 = W_mf.T.reshape(4, 8, _VW)
    Hmf3 = H_mf.T.reshape(4, 8, _VH)
    Wmlp_t = W_mlp.T[:, _VW - 128:]
    Hmlp_t = H_mlp.T[:, _VH - 128:]
    Wmf_t = W_mf.T[:, _VW - 128:]
    Hmf_t = H_mf.T[:, _VH - 128:]
    outs = _sc_gather(u, i, j, Wmlp3, Hmlp3, Wmf3, Hmf3,
                      Wmlp_t, Hmlp_t, Wmf_t, Hmf_t)
    return _tc_head(
        *outs,
        fc0_w.T, fc0_b.reshape(1, D),
        fc1_w.T, fc1_b.reshape(1, 16),
        aff_w.T, aff_b.reshape(1, 1))
